# baseline (speedup 1.0000x reference)
import functools

import jax
import jax.numpy as jnp
from jax.experimental import pallas as pl
from jax.experimental.pallas import tpu as pltpu

TOKENS = 16384
HID = 2048
NEXP = 64
K = 8
TM = 2048
TH = TM // 2


def _gate_body(xa_ref, xb_ref, w_ref, b_ref, idx_ref, wgt_ref):
    w = w_ref[...]
    la = jax.lax.dot_general(
        w, xa_ref[...], (((1,), (1,)), ((), ())), preferred_element_type=jnp.float32
    )
    lb = jax.lax.dot_general(
        w, xb_ref[...], (((1,), (1,)), ((), ())), preferred_element_type=jnp.float32
    )
    logits = jnp.concatenate([la, lb], axis=1)
    scores = jax.nn.sigmoid(logits)
    biased = scores + b_ref[...]
    iota = jax.lax.broadcasted_iota(jnp.int32, (NEXP, TM), 0)
    idxs, vals = [], []
    cur = biased
    for _ in range(K):
        m = jnp.max(cur, axis=0, keepdims=True)
        cand = jnp.where(cur == m, iota, NEXP)
        idx = jnp.min(cand, axis=0, keepdims=True)
        sel = cand == idx
        sval = jnp.sum(jnp.where(sel, scores, 0.0), axis=0, keepdims=True)
        cur = jnp.where(sel, -jnp.inf, cur)
        idxs.append(idx)
        vals.append(sval)
    topk_i = jnp.concatenate(idxs, axis=0)
    topk_v = jnp.concatenate(vals, axis=0)
    s = jnp.sum(topk_v, axis=0, keepdims=True) + 1e-20
    idx_ref[...] = topk_i.T
    wgt_ref[...] = (topk_v / s).T


@jax.jit
def kernel(x, W, e_score_correction_bias):
    bias2d = e_score_correction_bias.reshape(NEXP, 1)
    grid = (TOKENS // TM,)
    out_i, out_w = pl.pallas_call(
        _gate_body,
        grid=grid,
        in_specs=[
            pl.BlockSpec((TH, HID), lambda i: (2 * i, 0)),
            pl.BlockSpec((TH, HID), lambda i: (2 * i + 1, 0)),
            pl.BlockSpec((NEXP, HID), lambda i: (0, 0)),
            pl.BlockSpec((NEXP, 1), lambda i: (0, 0)),
        ],
        out_specs=[
            pl.BlockSpec((TM, K), lambda i: (i, 0)),
            pl.BlockSpec((TM, K), lambda i: (i, 0)),
        ],
        out_shape=[
            jax.ShapeDtypeStruct((TOKENS, K), jnp.int32),
            jax.ShapeDtypeStruct((TOKENS, K), jnp.float32),
        ],
        compiler_params=pltpu.CompilerParams(
            dimension_semantics=("parallel",),
        ),
    )(x, x, W, bias2d)
    return (out_i, out_w)
